# Initial kernel scaffold; baseline (speedup 1.0000x reference)
import jax
import jax.numpy as jnp
from jax.experimental import pallas as pl


def kernel(feat_prop, neigh_idx, W, b):
    raise NotImplementedError("write your pallas kernel here")


# speedup vs baseline: 1.2324x; 1.2324x over previous
import functools

import jax
import jax.numpy as jnp
from jax import lax
from jax.experimental import pallas as pl
from jax.experimental.pallas import tpu as pltpu
from jax.experimental.pallas import tpu_sc as plsc

_NC = 2
_NS = 16
_NW = _NC * _NS
_L = 16


def _rnorm_body(x_ref, o_ref):
    x = x_ref[...]
    o_ref[...] = lax.rsqrt(jnp.sum(x * x, axis=1))


def _final_body(k_inv, d, s_ref, ws_ref, x_ref, w_ref, b_ref, o_ref):
    wn = w_ref[:, :d]
    wc = w_ref[:, d:]
    dn = (((1,), (1,)), ((), ()))
    ctr = lax.dot_general(x_ref[...], wc, dn,
                          preferred_element_type=jnp.float32) + b_ref[...]
    sn = lax.dot_general(s_ref[...], wn, dn,
                         preferred_element_type=jnp.float32)
    o_ref[...] = (sn + ws_ref[...] * ctr) * k_inv


def _make_sc_gather(np_, c, k, d):
    mesh = plsc.VectorSubcoreMesh(core_axis_name="c", subcore_axis_name="s")
    nsub = d // _L

    kw = (k + 1 + _L - 1) // _L * _L

    def body(feat_hbm, idx_hbm, rn_hbm, s_hbm, ws_hbm,
             idx_v, rn_v, rows_v, s_v, ws_v, sem):
        wid = lax.axis_index("s") * _NC + lax.axis_index("c")
        base = wid * c
        pltpu.sync_copy(idx_hbm.at[pl.ds(base * kw, c * kw)], idx_v)
        pltpu.sync_copy(rn_hbm, rn_v)
        lane = lax.iota(jnp.int32, _L)

        def node(i, carry):
            ib = i * kw
            pltpu.async_copy(feat_hbm.at[idx_v.at[pl.ds(ib, k + 1)]],
                             rows_v, sem).wait()
            rnj = [plsc.load_gather(rn_v, [idx_v[pl.ds(ib + g * _L, _L)]])
                   for g in range(kw // _L)]
            rn_ctr = rnj[0][0]
            xn = [rows_v[0, pl.ds(t * _L, _L)] for t in range(nsub)]
            acc = [jnp.zeros((_L,), jnp.float32) for _ in range(nsub)]
            ws = jnp.float32(0.0)
            for e in range(1, k + 1):
                row = [rows_v[e, pl.ds(t * _L, _L)] for t in range(nsub)]
                dv = row[0] * xn[0]
                for t in range(1, nsub):
                    dv = dv + row[t] * xn[t]
                dot = jnp.sum(dv)
                w = dot * rnj[e // _L][e % _L] * rn_ctr
                ws = ws + w
                for t in range(nsub):
                    acc[t] = acc[t] + w * row[t]
            for t in range(nsub):
                s_v[i, pl.ds(t * _L, _L)] = acc[t]
            plsc.store_scatter(ws_v, [jnp.full((_L,), i, jnp.int32)],
                               jnp.full((_L,), ws, jnp.float32),
                               mask=lane == 0)
            return carry

        lax.fori_loop(0, c, node, 0, unroll=False)
        pltpu.sync_copy(s_v, s_hbm.at[pl.ds(base, c)])
        pltpu.sync_copy(ws_v, ws_hbm.at[pl.ds(base, c)])

    return pl.kernel(
        body,
        out_type=[
            jax.ShapeDtypeStruct((np_, d), jnp.float32),
            jax.ShapeDtypeStruct((np_,), jnp.float32),
        ],
        mesh=mesh,
        scratch_types=[
            pltpu.VMEM((c * kw,), jnp.int32),
            pltpu.VMEM((np_,), jnp.float32),
            pltpu.VMEM((k + 1, d), jnp.float32),
            pltpu.VMEM((c, d), jnp.float32),
            pltpu.VMEM((c,), jnp.float32),
            pltpu.SemaphoreType.DMA,
        ],
        compiler_params=pltpu.CompilerParams(needs_layout_passes=False),
    )


def kernel(feat_prop, neigh_idx, W, b):
    n, d = feat_prop.shape
    k = neigh_idx.shape[1]
    out_f = W.shape[0]
    c = (n + _NW - 1) // _NW
    c = (c + 7) // 8 * 8
    np_ = c * _NW

    kw = (k + 1 + _L - 1) // _L * _L
    xp = jnp.pad(feat_prop, ((0, np_ - n), (0, 0)))
    idxp = jnp.pad(neigh_idx.astype(jnp.int32), ((0, np_ - n), (0, 0)))
    idxe = jnp.concatenate(
        [jnp.arange(np_, dtype=jnp.int32)[:, None], idxp,
         jnp.zeros((np_, kw - k - 1), jnp.int32)], axis=1).reshape(np_ * kw)

    rnorm = pl.pallas_call(
        _rnorm_body,
        out_shape=jax.ShapeDtypeStruct((np_,), jnp.float32),
    )(xp)

    s, wsum = _make_sc_gather(np_, c, k, d)(xp, idxe, rnorm)
    wsum = wsum.reshape(np_, 1)

    blk = 1024
    grid = np_ // blk
    out = pl.pallas_call(
        functools.partial(_final_body, 1.0 / k, d),
        grid=(grid,),
        in_specs=[
            pl.BlockSpec((blk, d), lambda i: (i, 0)),
            pl.BlockSpec((blk, 1), lambda i: (i, 0)),
            pl.BlockSpec((blk, d), lambda i: (i, 0)),
            pl.BlockSpec((out_f, 2 * d), lambda i: (0, 0)),
            pl.BlockSpec((out_f,), lambda i: (0,)),
        ],
        out_specs=pl.BlockSpec((blk, out_f), lambda i: (i, 0)),
        out_shape=jax.ShapeDtypeStruct((np_, out_f), jnp.float32),
    )(s, wsum, xp, W, b)

    return out[:n]
